# Initial kernel scaffold; baseline (speedup 1.0000x reference)
import jax
import jax.numpy as jnp
from jax.experimental import pallas as pl


def kernel(x, edge_index, W1, b1, W2, b2, Wfc, bfc):
    raise NotImplementedError("write your pallas kernel here")


# speedup vs baseline: 26.9047x; 26.9047x over previous
import functools

import jax
import jax.numpy as jnp
from jax import lax
from jax.experimental import pallas as pl
from jax.experimental.pallas import tpu as pltpu
from jax.experimental.pallas import tpu_sc as plsc

NC = 2
NS = 16
NW = NC * NS
CHUNK = 80


def _mesh():
    return plsc.VectorSubcoreMesh(
        core_axis_name="c", subcore_axis_name="s", num_cores=NC, num_subcores=NS
    )


_SC_PARAMS = pltpu.CompilerParams(use_tc_tiling_on_sc=False)


def _zero_shared(zbuf, acc, rows_per_tile, d, sid):
    def zstore(r, _):
        for j in range(d // 16):
            zbuf[r, pl.ds(j * 16, 16)] = jnp.zeros((16,), jnp.float32)
        return 0

    lax.fori_loop(0, rows_per_tile, zstore, 0)
    pltpu.sync_copy(zbuf, acc.at[pl.ds(sid * rows_per_tile, rows_per_tile)])


def _copy_small(src_all, buf, off, n):
    for j in range(n // 16):
        buf[pl.ds(j * 16, 16)] = src_all[pl.ds(off + j * 16, 16)]


def _make_deg_kernel(n_pad, e):
    e_per_w = e // NW
    assert e_per_w % CHUNK == 0
    rows_per_tile = n_pad // NS

    @functools.partial(
        pl.kernel,
        out_type=jax.ShapeDtypeStruct((NC, n_pad, 16), jnp.float32),
        mesh=_mesh(),
        scratch_types=[
            pltpu.VMEM((e_per_w,), jnp.int32),
            pltpu.VMEM((CHUNK,), jnp.int32),
            pltpu.VMEM((CHUNK, 16), jnp.float32),
            pltpu.VMEM((rows_per_tile, 16), jnp.float32),
            pltpu.VMEM_SHARED((n_pad, 16), jnp.float32),
        ],
        compiler_params=_SC_PARAMS,
    )
    def deg_kernel(dst_hbm, out_hbm, dst_all, dstbuf, ones, zbuf, acc):
        cid = lax.axis_index("c")
        sid = lax.axis_index("s")
        wid = cid * NS + sid
        base = wid * e_per_w

        def fill_ones(r, _):
            ones[r, pl.ds(0, 16)] = jnp.ones((16,), jnp.float32)
            return 0

        lax.fori_loop(0, CHUNK, fill_ones, 0)
        _zero_shared(zbuf, acc, rows_per_tile, 16, sid)
        pltpu.sync_copy(dst_hbm.at[pl.ds(base, e_per_w)], dst_all)
        plsc.subcore_barrier()

        def body(i, _):
            _copy_small(dst_all, dstbuf, i * CHUNK, CHUNK)
            pltpu.sync_copy(ones, acc.at[dstbuf], add=True)
            return 0

        lax.fori_loop(0, e_per_w // CHUNK, body, 0)
        plsc.subcore_barrier()
        pltpu.sync_copy(
            acc.at[pl.ds(sid * rows_per_tile, rows_per_tile)],
            out_hbm.at[cid, pl.ds(sid * rows_per_tile, rows_per_tile)],
        )

    return deg_kernel


def _make_scatter_kernel(n, n_pad, e, d):
    e_per_w = e // NW
    assert e_per_w % CHUNK == 0
    rows_per_tile = n_pad // NS

    @functools.partial(
        pl.kernel,
        out_type=jax.ShapeDtypeStruct((NC, n_pad, d), jnp.float32),
        mesh=_mesh(),
        scratch_types=[
            pltpu.VMEM((e_per_w,), jnp.int32),
            pltpu.VMEM((e_per_w,), jnp.int32),
            pltpu.VMEM((CHUNK,), jnp.int32),
            pltpu.VMEM((CHUNK,), jnp.int32),
            pltpu.VMEM((CHUNK, d), jnp.float32),
            pltpu.VMEM((rows_per_tile, d), jnp.float32),
            pltpu.VMEM_SHARED((n_pad, d), jnp.float32),
        ],
        compiler_params=_SC_PARAMS,
    )
    def scatter_kernel(
        g_hbm, src_hbm, dst_hbm, out_hbm,
        src_all, dst_all, srcbuf, dstbuf, rows, zbuf, acc,
    ):
        cid = lax.axis_index("c")
        sid = lax.axis_index("s")
        wid = cid * NS + sid
        base = wid * e_per_w

        _zero_shared(zbuf, acc, rows_per_tile, d, sid)
        pltpu.sync_copy(src_hbm.at[pl.ds(base, e_per_w)], src_all)
        pltpu.sync_copy(dst_hbm.at[pl.ds(base, e_per_w)], dst_all)
        plsc.subcore_barrier()

        def body(i, _):
            off = i * CHUNK
            _copy_small(src_all, srcbuf, off, CHUNK)
            _copy_small(dst_all, dstbuf, off, CHUNK)
            pltpu.sync_copy(g_hbm.at[srcbuf], rows)
            pltpu.sync_copy(rows, acc.at[dstbuf], add=True)
            return 0

        lax.fori_loop(0, e_per_w // CHUNK, body, 0)
        plsc.subcore_barrier()
        pltpu.sync_copy(
            acc.at[pl.ds(sid * rows_per_tile, rows_per_tile)],
            out_hbm.at[cid, pl.ds(sid * rows_per_tile, rows_per_tile)],
        )

    return scatter_kernel


def _mm1_body(x_ref, w_ref, o_ref):
    o_ref[...] = jnp.dot(x_ref[...], w_ref[...], preferred_element_type=jnp.float32)


def _scale1_body(h1_ref, degp_ref, g1_ref, dinv_ref):
    nrows = h1_ref.shape[0]
    dp = degp_ref[...]
    deg = 1.0 + dp[0, :nrows, 0:1] + dp[1, :nrows, 0:1]
    dinv = lax.rsqrt(deg)
    dinv_ref[...] = dinv
    g1_ref[...] = h1_ref[...] * dinv


def _mid_body(sp_ref, g_ref, dinv_ref, b_ref, w_ref, o_ref):
    nrows = g_ref.shape[0]
    p = sp_ref[...]
    s = g_ref[...] + p[0, :nrows] + p[1, :nrows]
    dinv = dinv_ref[...]
    a = jnp.maximum(s * dinv + b_ref[...], 0.0)
    o_ref[...] = jnp.dot(a, w_ref[...], preferred_element_type=jnp.float32) * dinv


def _out_body(sp_ref, g_ref, dinv_ref, b_ref, wfc_ref, bfc_ref, o_ref):
    nrows = g_ref.shape[0]
    p = sp_ref[...]
    s = g_ref[...] + p[0, :nrows] + p[1, :nrows]
    a = jnp.maximum(s * dinv_ref[...] + b_ref[...], 0.0)
    h = jnp.dot(a, wfc_ref[...], preferred_element_type=jnp.float32) + bfc_ref[...]
    m = jnp.max(h, axis=1, keepdims=True)
    lse = m + jnp.log(jnp.sum(jnp.exp(h - m), axis=1, keepdims=True))
    o_ref[...] = h - lse


def _tc_call(body, out_shape, *args):
    return pl.pallas_call(body, out_shape=out_shape)(*args)


def kernel(x, edge_index, W1, b1, W2, b2, Wfc, bfc):
    n, _ = x.shape
    e = edge_index.shape[1]
    src = edge_index[0].astype(jnp.int32)
    dst = edge_index[1].astype(jnp.int32)
    d1 = W1.shape[1]
    d2 = W2.shape[1]

    f32 = jnp.float32
    n_pad = ((n + 127) // 128) * 128
    degp = _make_deg_kernel(n_pad, e)(dst)
    h1 = _tc_call(_mm1_body, jax.ShapeDtypeStruct((n, d1), f32), x, W1)
    g1, dinv = pl.pallas_call(
        _scale1_body,
        out_shape=(
            jax.ShapeDtypeStruct((n, d1), f32),
            jax.ShapeDtypeStruct((n, 1), f32),
        ),
    )(h1, degp)
    s1p = _make_scatter_kernel(n, n_pad, e, d1)(g1, src, dst)
    g2 = _tc_call(
        _mid_body, jax.ShapeDtypeStruct((n, d2), f32),
        s1p, g1, dinv, b1.reshape(1, d1), W2,
    )
    s2p = _make_scatter_kernel(n, n_pad, e, d2)(g2, src, dst)
    out = _tc_call(
        _out_body, jax.ShapeDtypeStruct((n, 2), f32),
        s2p, g2, dinv, b2.reshape(1, d2), Wfc, bfc.reshape(1, 2),
    )
    return out


# speedup vs baseline: 50.2988x; 1.8695x over previous
import functools

import jax
import jax.numpy as jnp
from jax import lax
from jax.experimental import pallas as pl
from jax.experimental.pallas import tpu as pltpu
from jax.experimental.pallas import tpu_sc as plsc

NC = 2
NS = 16
NW = NC * NS
CHUNK = 80
NBUF = 5


def _mesh():
    return plsc.VectorSubcoreMesh(
        core_axis_name="c", subcore_axis_name="s", num_cores=NC, num_subcores=NS
    )


_SC_PARAMS = pltpu.CompilerParams(use_tc_tiling_on_sc=False)


def _zero_shared(zbuf, acc, rows_per_tile, d, sid):
    def zstore(r, _):
        for j in range(d // 16):
            zbuf[r, pl.ds(j * 16, 16)] = jnp.zeros((16,), jnp.float32)
        return 0

    lax.fori_loop(0, rows_per_tile, zstore, 0)
    pltpu.sync_copy(zbuf, acc.at[pl.ds(sid * rows_per_tile, rows_per_tile)])


def _copy_small(src_all, buf, off, n):
    for j in range(n // 16):
        buf[pl.ds(j * 16, 16)] = src_all[pl.ds(off + j * 16, 16)]


def _make_deg_kernel(n_pad, e):
    e_per_w = e // NW
    assert e_per_w % (CHUNK * NBUF) == 0
    rows_per_tile = n_pad // NS

    @functools.partial(
        pl.kernel,
        out_type=jax.ShapeDtypeStruct((NC, n_pad, 16), jnp.float32),
        mesh=_mesh(),
        scratch_types=[
            pltpu.VMEM((e_per_w,), jnp.int32),
            [pltpu.VMEM((CHUNK,), jnp.int32) for _ in range(NBUF)],
            pltpu.VMEM((CHUNK, 16), jnp.float32),
            pltpu.VMEM((rows_per_tile, 16), jnp.float32),
            pltpu.VMEM_SHARED((n_pad, 16), jnp.float32),
            [pltpu.SemaphoreType.DMA for _ in range(NBUF)],
        ],
        compiler_params=_SC_PARAMS,
    )
    def deg_kernel(dst_hbm, out_hbm, dst_all, dstbuf, ones, zbuf, acc, ssem):
        cid = lax.axis_index("c")
        sid = lax.axis_index("s")
        wid = cid * NS + sid
        base = wid * e_per_w

        def fill_ones(r, _):
            ones[r, pl.ds(0, 16)] = jnp.ones((16,), jnp.float32)
            return 0

        lax.fori_loop(0, CHUNK, fill_ones, 0)
        _zero_shared(zbuf, acc, rows_per_tile, 16, sid)
        pltpu.sync_copy(dst_hbm.at[pl.ds(base, e_per_w)], dst_all)
        plsc.subcore_barrier()

        def outer(o, _):
            for b in range(NBUF):
                off = (o * NBUF + b) * CHUNK

                @pl.when(o > 0)
                def _wait_prev_scatter():
                    pltpu.make_async_copy(ones, acc.at[dstbuf[b]], ssem[b]).wait()

                _copy_small(dst_all, dstbuf[b], off, CHUNK)
                pltpu.async_copy(ones, acc.at[dstbuf[b]], ssem[b], add=True)
            return 0

        lax.fori_loop(0, e_per_w // (CHUNK * NBUF), outer, 0)
        for b in range(NBUF):
            pltpu.make_async_copy(ones, acc.at[dstbuf[b]], ssem[b]).wait()
        plsc.subcore_barrier()
        pltpu.sync_copy(
            acc.at[pl.ds(sid * rows_per_tile, rows_per_tile)],
            out_hbm.at[cid, pl.ds(sid * rows_per_tile, rows_per_tile)],
        )

    return deg_kernel


def _make_scatter_kernel(n, n_pad, e, d):
    e_per_w = e // NW
    assert e_per_w % (CHUNK * NBUF) == 0
    rows_per_tile = n_pad // NS

    @functools.partial(
        pl.kernel,
        out_type=jax.ShapeDtypeStruct((NC, n_pad, d), jnp.float32),
        mesh=_mesh(),
        scratch_types=[
            pltpu.VMEM((e_per_w,), jnp.int32),
            pltpu.VMEM((e_per_w,), jnp.int32),
            [pltpu.VMEM((CHUNK,), jnp.int32) for _ in range(NBUF)],
            [pltpu.VMEM((CHUNK, d), jnp.float32) for _ in range(NBUF)],
            pltpu.VMEM((rows_per_tile, d), jnp.float32),
            pltpu.VMEM_SHARED((n_pad, d), jnp.float32),
            [pltpu.SemaphoreType.DMA for _ in range(NBUF)],
            [pltpu.SemaphoreType.DMA for _ in range(NBUF)],
        ],
        compiler_params=_SC_PARAMS,
    )
    def scatter_kernel(
        g_hbm, src_hbm, dst_hbm, out_hbm,
        src_all, dst_all, dstbuf, rows, zbuf, acc, gsem, ssem,
    ):
        cid = lax.axis_index("c")
        sid = lax.axis_index("s")
        wid = cid * NS + sid
        base = wid * e_per_w

        _zero_shared(zbuf, acc, rows_per_tile, d, sid)
        pltpu.sync_copy(src_hbm.at[pl.ds(base, e_per_w)], src_all)
        pltpu.sync_copy(dst_hbm.at[pl.ds(base, e_per_w)], dst_all)
        plsc.subcore_barrier()

        def outer(o, _):
            for b in range(NBUF):
                off = (o * NBUF + b) * CHUNK

                @pl.when(o > 0)
                def _wait_prev_scatter():
                    pltpu.make_async_copy(rows[b], acc.at[dstbuf[b]], ssem[b]).wait()

                _copy_small(dst_all, dstbuf[b], off, CHUNK)
                pltpu.async_copy(
                    g_hbm.at[src_all.at[pl.ds(off, CHUNK)]], rows[b], gsem[b]
                )
            for b in range(NBUF):
                off = (o * NBUF + b) * CHUNK
                pltpu.make_async_copy(
                    g_hbm.at[src_all.at[pl.ds(off, CHUNK)]], rows[b], gsem[b]
                ).wait()
                pltpu.async_copy(rows[b], acc.at[dstbuf[b]], ssem[b], add=True)
            return 0

        lax.fori_loop(0, e_per_w // (CHUNK * NBUF), outer, 0)
        for b in range(NBUF):
            pltpu.make_async_copy(rows[b], acc.at[dstbuf[b]], ssem[b]).wait()
        plsc.subcore_barrier()
        pltpu.sync_copy(
            acc.at[pl.ds(sid * rows_per_tile, rows_per_tile)],
            out_hbm.at[cid, pl.ds(sid * rows_per_tile, rows_per_tile)],
        )

    return scatter_kernel


def _mm1_body(x_ref, w_ref, o_ref):
    o_ref[...] = jnp.dot(x_ref[...], w_ref[...], preferred_element_type=jnp.float32)


def _scale1_body(h1_ref, degp_ref, g1_ref, dinv_ref):
    nrows = h1_ref.shape[0]
    dp = degp_ref[...]
    deg = 1.0 + dp[0, :nrows, 0:1] + dp[1, :nrows, 0:1]
    dinv = lax.rsqrt(deg)
    dinv_ref[...] = dinv
    g1_ref[...] = h1_ref[...] * dinv


def _mid_body(sp_ref, g_ref, dinv_ref, b_ref, w_ref, o_ref):
    nrows = g_ref.shape[0]
    p = sp_ref[...]
    s = g_ref[...] + p[0, :nrows] + p[1, :nrows]
    dinv = dinv_ref[...]
    a = jnp.maximum(s * dinv + b_ref[...], 0.0)
    o_ref[...] = jnp.dot(a, w_ref[...], preferred_element_type=jnp.float32) * dinv


def _out_body(sp_ref, g_ref, dinv_ref, b_ref, wfc_ref, bfc_ref, o_ref):
    nrows = g_ref.shape[0]
    p = sp_ref[...]
    s = g_ref[...] + p[0, :nrows] + p[1, :nrows]
    a = jnp.maximum(s * dinv_ref[...] + b_ref[...], 0.0)
    h = jnp.dot(a, wfc_ref[...], preferred_element_type=jnp.float32) + bfc_ref[...]
    m = jnp.max(h, axis=1, keepdims=True)
    lse = m + jnp.log(jnp.sum(jnp.exp(h - m), axis=1, keepdims=True))
    o_ref[...] = h - lse


def _tc_call(body, out_shape, *args):
    return pl.pallas_call(body, out_shape=out_shape)(*args)


def kernel(x, edge_index, W1, b1, W2, b2, Wfc, bfc):
    n, _ = x.shape
    e = edge_index.shape[1]
    src = edge_index[0].astype(jnp.int32)
    dst = edge_index[1].astype(jnp.int32)
    d1 = W1.shape[1]
    d2 = W2.shape[1]

    f32 = jnp.float32
    n_pad = ((n + 127) // 128) * 128
    degp = _make_deg_kernel(n_pad, e)(dst)
    h1 = _tc_call(_mm1_body, jax.ShapeDtypeStruct((n, d1), f32), x, W1)
    g1, dinv = pl.pallas_call(
        _scale1_body,
        out_shape=(
            jax.ShapeDtypeStruct((n, d1), f32),
            jax.ShapeDtypeStruct((n, 1), f32),
        ),
    )(h1, degp)
    s1p = _make_scatter_kernel(n, n_pad, e, d1)(g1, src, dst)
    g2 = _tc_call(
        _mid_body, jax.ShapeDtypeStruct((n, d2), f32),
        s1p, g1, dinv, b1.reshape(1, d1), W2,
    )
    s2p = _make_scatter_kernel(n, n_pad, e, d2)(g2, src, dst)
    out = _tc_call(
        _out_body, jax.ShapeDtypeStruct((n, 2), f32),
        s2p, g2, dinv, b2.reshape(1, d2), Wfc, bfc.reshape(1, 2),
    )
    return out


# speedup vs baseline: 50.7370x; 1.0087x over previous
import functools

import jax
import jax.numpy as jnp
from jax import lax
from jax.experimental import pallas as pl
from jax.experimental.pallas import tpu as pltpu
from jax.experimental.pallas import tpu_sc as plsc

NC = 2
NS = 16
NW = NC * NS
CHUNK = 80
NBUF = 5


def _mesh():
    return plsc.VectorSubcoreMesh(
        core_axis_name="c", subcore_axis_name="s", num_cores=NC, num_subcores=NS
    )


_SC_PARAMS = pltpu.CompilerParams(use_tc_tiling_on_sc=False)


def _zero_shared(zbuf, acc, rows_per_tile, d, sid):
    def zstore(r, _):
        for j in range(d // 16):
            zbuf[r, pl.ds(j * 16, 16)] = jnp.zeros((16,), jnp.float32)
        return 0

    lax.fori_loop(0, rows_per_tile, zstore, 0)
    pltpu.sync_copy(zbuf, acc.at[pl.ds(sid * rows_per_tile, rows_per_tile)])


def _copy_small(src_all, buf, off, n):
    for j in range(n // 16):
        buf[pl.ds(j * 16, 16)] = src_all[pl.ds(off + j * 16, 16)]


def _make_deg_kernel(n_pad, e):
    e_per_w = e // NW
    assert e_per_w % (CHUNK * NBUF) == 0
    rows_per_tile = n_pad // NS

    @functools.partial(
        pl.kernel,
        out_type=jax.ShapeDtypeStruct((NC, n_pad, 16), jnp.float32),
        mesh=_mesh(),
        scratch_types=[
            pltpu.VMEM((e_per_w,), jnp.int32),
            [pltpu.VMEM((CHUNK,), jnp.int32) for _ in range(NBUF)],
            pltpu.VMEM((CHUNK, 16), jnp.float32),
            pltpu.VMEM((rows_per_tile, 16), jnp.float32),
            pltpu.VMEM_SHARED((n_pad, 16), jnp.float32),
            [pltpu.SemaphoreType.DMA for _ in range(NBUF)],
        ],
        compiler_params=_SC_PARAMS,
    )
    def deg_kernel(dst_hbm, out_hbm, dst_all, dstbuf, ones, zbuf, acc, ssem):
        cid = lax.axis_index("c")
        sid = lax.axis_index("s")
        wid = cid * NS + sid
        base = wid * e_per_w

        def fill_ones(r, _):
            ones[r, pl.ds(0, 16)] = jnp.ones((16,), jnp.float32)
            return 0

        lax.fori_loop(0, CHUNK, fill_ones, 0)
        _zero_shared(zbuf, acc, rows_per_tile, 16, sid)
        pltpu.sync_copy(dst_hbm.at[pl.ds(base, e_per_w)], dst_all)
        plsc.subcore_barrier()

        def outer(o, _):
            for b in range(NBUF):
                off = (o * NBUF + b) * CHUNK

                @pl.when(o > 0)
                def _wait_prev_scatter():
                    pltpu.make_async_copy(ones, acc.at[dstbuf[b]], ssem[b]).wait()

                _copy_small(dst_all, dstbuf[b], off, CHUNK)
                pltpu.async_copy(ones, acc.at[dstbuf[b]], ssem[b], add=True)
            return 0

        lax.fori_loop(0, e_per_w // (CHUNK * NBUF), outer, 0)
        for b in range(NBUF):
            pltpu.make_async_copy(ones, acc.at[dstbuf[b]], ssem[b]).wait()
        plsc.subcore_barrier()
        pltpu.sync_copy(
            acc.at[pl.ds(sid * rows_per_tile, rows_per_tile)],
            out_hbm.at[cid, pl.ds(sid * rows_per_tile, rows_per_tile)],
        )

    return deg_kernel


def _make_scatter_kernel(n, n_pad, e, d):
    e_per_w = e // NW
    assert e_per_w % (CHUNK * NBUF) == 0
    rows_per_tile = n_pad // NS

    nch = e_per_w // CHUNK
    scratch_types = [
        pltpu.VMEM((nch, CHUNK), jnp.int32),
        pltpu.VMEM((nch, CHUNK), jnp.int32),
        [pltpu.VMEM((CHUNK, d), jnp.float32) for _ in range(NBUF)],
        pltpu.VMEM((rows_per_tile, d), jnp.float32),
        pltpu.VMEM_SHARED((n_pad, d), jnp.float32),
        [pltpu.SemaphoreType.DMA for _ in range(NBUF)],
        [pltpu.SemaphoreType.DMA for _ in range(NBUF)],
    ]

    @functools.partial(
        pl.kernel,
        out_type=jax.ShapeDtypeStruct((NC, n_pad, d), jnp.float32),
        mesh=_mesh(),
        scratch_types=scratch_types,
        compiler_params=_SC_PARAMS,
    )
    def scatter_kernel(
        g_hbm, src_hbm, dst_hbm, out_hbm,
        src_all, dst_all, rows, zbuf, acc, gsem, ssem,
    ):
        cid = lax.axis_index("c")
        sid = lax.axis_index("s")
        wid = cid * NS + sid
        base = wid * nch

        _zero_shared(zbuf, acc, rows_per_tile, d, sid)
        pltpu.sync_copy(src_hbm.at[pl.ds(base, nch)], src_all)
        pltpu.sync_copy(dst_hbm.at[pl.ds(base, nch)], dst_all)
        plsc.subcore_barrier()

        def outer(o, _):
            for b in range(NBUF):
                i = o * NBUF + b

                @pl.when(o > 0)
                def _wait_prev_scatter():
                    pltpu.make_async_copy(
                        rows[b], acc.at[dst_all.at[i - NBUF]], ssem[b]
                    ).wait()

                pltpu.async_copy(g_hbm.at[src_all.at[i]], rows[b], gsem[b])
            for b in range(NBUF):
                i = o * NBUF + b
                pltpu.make_async_copy(
                    g_hbm.at[src_all.at[i]], rows[b], gsem[b]
                ).wait()
                pltpu.async_copy(rows[b], acc.at[dst_all.at[i]], ssem[b], add=True)
            return 0

        lax.fori_loop(0, nch // NBUF, outer, 0)
        for b in range(NBUF):
            i = nch - NBUF + b
            pltpu.make_async_copy(rows[b], acc.at[dst_all.at[i]], ssem[b]).wait()
        plsc.subcore_barrier()
        pltpu.sync_copy(
            acc.at[pl.ds(sid * rows_per_tile, rows_per_tile)],
            out_hbm.at[cid, pl.ds(sid * rows_per_tile, rows_per_tile)],
        )

    return scatter_kernel


def _lin1_body(x_ref, w_ref, degp_ref, g1_ref, dinv_ref):
    nrows = x_ref.shape[0]
    dp = degp_ref[...]
    deg = 1.0 + dp[0, :nrows, 0:1] + dp[1, :nrows, 0:1]
    dinv = lax.rsqrt(deg)
    dinv_ref[...] = dinv
    h1 = jnp.dot(x_ref[...], w_ref[...], preferred_element_type=jnp.float32)
    g1_ref[...] = h1 * dinv


def _mid_body(sp_ref, g_ref, dinv_ref, b_ref, w_ref, o_ref):
    nrows = g_ref.shape[0]
    p = sp_ref[...]
    s = g_ref[...] + p[0, :nrows] + p[1, :nrows]
    dinv = dinv_ref[...]
    a = jnp.maximum(s * dinv + b_ref[...], 0.0)
    o_ref[...] = jnp.dot(a, w_ref[...], preferred_element_type=jnp.float32) * dinv


def _out_body(sp_ref, g_ref, dinv_ref, b_ref, wfc_ref, bfc_ref, o_ref):
    nrows = g_ref.shape[0]
    p = sp_ref[...]
    s = g_ref[...] + p[0, :nrows] + p[1, :nrows]
    a = jnp.maximum(s * dinv_ref[...] + b_ref[...], 0.0)
    h = jnp.dot(a, wfc_ref[...], preferred_element_type=jnp.float32) + bfc_ref[...]
    m = jnp.max(h, axis=1, keepdims=True)
    lse = m + jnp.log(jnp.sum(jnp.exp(h - m), axis=1, keepdims=True))
    o_ref[...] = h - lse


def _tc_call(body, out_shape, *args):
    return pl.pallas_call(body, out_shape=out_shape)(*args)


def kernel(x, edge_index, W1, b1, W2, b2, Wfc, bfc):
    n, _ = x.shape
    e = edge_index.shape[1]
    src = edge_index[0].astype(jnp.int32)
    dst = edge_index[1].astype(jnp.int32)
    d1 = W1.shape[1]
    d2 = W2.shape[1]

    f32 = jnp.float32
    n_pad = ((n + 127) // 128) * 128
    src2d = src.reshape(e // CHUNK, CHUNK)
    dst2d = dst.reshape(e // CHUNK, CHUNK)
    degp = _make_deg_kernel(n_pad, e)(dst)
    g1, dinv = pl.pallas_call(
        _lin1_body,
        out_shape=(
            jax.ShapeDtypeStruct((n, d1), f32),
            jax.ShapeDtypeStruct((n, 1), f32),
        ),
    )(x, W1, degp)
    s1p = _make_scatter_kernel(n, n_pad, e, d1)(g1, src2d, dst2d)
    g2 = _tc_call(
        _mid_body, jax.ShapeDtypeStruct((n, d2), f32),
        s1p, g1, dinv, b1.reshape(1, d1), W2,
    )
    s2p = _make_scatter_kernel(n, n_pad, e, d2)(g2, src2d, dst2d)
    out = _tc_call(
        _out_body, jax.ShapeDtypeStruct((n, 2), f32),
        s2p, g2, dinv, b2.reshape(1, d2), Wfc, bfc.reshape(1, 2),
    )
    return out


# speedup vs baseline: 54.4588x; 1.0734x over previous
import functools

import jax
import jax.numpy as jnp
from jax import lax
from jax.experimental import pallas as pl
from jax.experimental.pallas import tpu as pltpu
from jax.experimental.pallas import tpu_sc as plsc

NC = 2
NS = 16
NW = NC * NS
CHUNK = 80
NBUF = 5


def _mesh():
    return plsc.VectorSubcoreMesh(
        core_axis_name="c", subcore_axis_name="s", num_cores=NC, num_subcores=NS
    )


_SC_PARAMS = pltpu.CompilerParams(use_tc_tiling_on_sc=False)


def _zero_shared(zbuf, acc, rows_per_tile, d, sid):
    def zstore(r, _):
        for j in range(d // 16):
            zbuf[r, pl.ds(j * 16, 16)] = jnp.zeros((16,), jnp.float32)
        return 0

    lax.fori_loop(0, rows_per_tile, zstore, 0)
    pltpu.sync_copy(zbuf, acc.at[pl.ds(sid * rows_per_tile, rows_per_tile)])


def _copy_small(src_all, buf, off, n):
    for j in range(n // 16):
        buf[pl.ds(j * 16, 16)] = src_all[pl.ds(off + j * 16, 16)]


def _make_deg_kernel(n_pad, e):
    e_per_w = e // NW
    assert e_per_w % (CHUNK * NBUF) == 0
    nch = e_per_w // CHUNK
    rows_per_tile = n_pad // NS

    @functools.partial(
        pl.kernel,
        out_type=jax.ShapeDtypeStruct((NC, n_pad, 16), jnp.float32),
        mesh=_mesh(),
        scratch_types=[
            pltpu.VMEM((nch, CHUNK), jnp.int32),
            pltpu.VMEM((CHUNK, 16), jnp.float32),
            pltpu.VMEM((rows_per_tile, 16), jnp.float32),
            pltpu.VMEM_SHARED((n_pad, 16), jnp.float32),
            [pltpu.SemaphoreType.DMA for _ in range(NBUF)],
        ],
        compiler_params=_SC_PARAMS,
    )
    def deg_kernel(ei_hbm, out_hbm, dst_all, ones, zbuf, acc, ssem):
        cid = lax.axis_index("c")
        sid = lax.axis_index("s")
        wid = cid * NS + sid
        base = wid * nch

        def fill_ones(r, _):
            ones[r, pl.ds(0, 16)] = jnp.ones((16,), jnp.float32)
            return 0

        lax.fori_loop(0, CHUNK, fill_ones, 0)
        _zero_shared(zbuf, acc, rows_per_tile, 16, sid)
        pltpu.sync_copy(ei_hbm.at[1, pl.ds(base, nch)], dst_all)
        plsc.subcore_barrier()

        def outer(o, _):
            for b in range(NBUF):
                i = o * NBUF + b

                @pl.when(o > 0)
                def _wait_prev_scatter():
                    pltpu.make_async_copy(
                        ones, acc.at[dst_all.at[i - NBUF]], ssem[b]
                    ).wait()

                pltpu.async_copy(ones, acc.at[dst_all.at[i]], ssem[b], add=True)
            return 0

        lax.fori_loop(0, nch // NBUF, outer, 0)
        for b in range(NBUF):
            i = nch - NBUF + b
            pltpu.make_async_copy(ones, acc.at[dst_all.at[i]], ssem[b]).wait()
        plsc.subcore_barrier()
        pltpu.sync_copy(
            acc.at[pl.ds(sid * rows_per_tile, rows_per_tile)],
            out_hbm.at[cid, pl.ds(sid * rows_per_tile, rows_per_tile)],
        )

    return deg_kernel


def _make_scatter_kernel(n, n_pad, e, d):
    e_per_w = e // NW
    assert e_per_w % (CHUNK * NBUF) == 0
    rows_per_tile = n_pad // NS

    nch = e_per_w // CHUNK
    scratch_types = [
        pltpu.VMEM((nch, CHUNK), jnp.int32),
        pltpu.VMEM((nch, CHUNK), jnp.int32),
        [pltpu.VMEM((CHUNK, d), jnp.float32) for _ in range(NBUF)],
        pltpu.VMEM((rows_per_tile, d), jnp.float32),
        pltpu.VMEM_SHARED((n_pad, d), jnp.float32),
        [pltpu.SemaphoreType.DMA for _ in range(NBUF)],
        [pltpu.SemaphoreType.DMA for _ in range(NBUF)],
    ]

    @functools.partial(
        pl.kernel,
        out_type=jax.ShapeDtypeStruct((NC, n_pad, d), jnp.float32),
        mesh=_mesh(),
        scratch_types=scratch_types,
        compiler_params=_SC_PARAMS,
    )
    def scatter_kernel(
        g_hbm, ei_hbm, out_hbm,
        src_all, dst_all, rows, zbuf, acc, gsem, ssem,
    ):
        cid = lax.axis_index("c")
        sid = lax.axis_index("s")
        wid = cid * NS + sid
        base = wid * nch

        _zero_shared(zbuf, acc, rows_per_tile, d, sid)
        pltpu.sync_copy(ei_hbm.at[0, pl.ds(base, nch)], src_all)
        pltpu.sync_copy(ei_hbm.at[1, pl.ds(base, nch)], dst_all)
        plsc.subcore_barrier()

        def outer(o, _):
            for b in range(NBUF):
                i = o * NBUF + b

                @pl.when(o > 0)
                def _wait_prev_scatter():
                    pltpu.make_async_copy(
                        rows[b], acc.at[dst_all.at[i - NBUF]], ssem[b]
                    ).wait()

                pltpu.async_copy(g_hbm.at[src_all.at[i]], rows[b], gsem[b])
            for b in range(NBUF):
                i = o * NBUF + b
                pltpu.make_async_copy(
                    g_hbm.at[src_all.at[i]], rows[b], gsem[b]
                ).wait()
                pltpu.async_copy(rows[b], acc.at[dst_all.at[i]], ssem[b], add=True)
            return 0

        lax.fori_loop(0, nch // NBUF, outer, 0)
        for b in range(NBUF):
            i = nch - NBUF + b
            pltpu.make_async_copy(rows[b], acc.at[dst_all.at[i]], ssem[b]).wait()
        plsc.subcore_barrier()
        pltpu.sync_copy(
            acc.at[pl.ds(sid * rows_per_tile, rows_per_tile)],
            out_hbm.at[cid, pl.ds(sid * rows_per_tile, rows_per_tile)],
        )

    return scatter_kernel


BLK = 2000


def _lin1_body(x_ref, w_ref, degp_ref, g1_ref, dinv_ref):
    dp = degp_ref[...]
    deg = 1.0 + dp[0, :, 0:1] + dp[1, :, 0:1]
    dinv = lax.rsqrt(deg)
    dinv_ref[...] = dinv
    h1 = jnp.dot(x_ref[...], w_ref[...], preferred_element_type=jnp.float32)
    g1_ref[...] = h1 * dinv


def _mid_body(sp_ref, g_ref, dinv_ref, b_ref, w_ref, o_ref):
    p = sp_ref[...]
    s = g_ref[...] + p[0] + p[1]
    dinv = dinv_ref[...]
    a = jnp.maximum(s * dinv + b_ref[...], 0.0)
    o_ref[...] = jnp.dot(a, w_ref[...], preferred_element_type=jnp.float32) * dinv


def _out_body(sp_ref, g_ref, dinv_ref, b_ref, wfc_ref, bfc_ref, o_ref):
    p = sp_ref[...]
    s = g_ref[...] + p[0] + p[1]
    a = jnp.maximum(s * dinv_ref[...] + b_ref[...], 0.0)
    h = jnp.dot(a, wfc_ref[...], preferred_element_type=jnp.float32) + bfc_ref[...]
    m = jnp.max(h, axis=1, keepdims=True)
    lse = m + jnp.log(jnp.sum(jnp.exp(h - m), axis=1, keepdims=True))
    o_ref[...] = h - lse


def _rows(shape):
    if len(shape) == 2:
        return pl.BlockSpec((BLK, shape[1]), lambda i: (i, 0))
    return pl.BlockSpec((shape[0], BLK, shape[2]), lambda i: (0, i, 0))


def _full(shape):
    return pl.BlockSpec(shape, lambda i: tuple(0 for _ in shape))


def kernel(x, edge_index, W1, b1, W2, b2, Wfc, bfc):
    n, d_in = x.shape
    e = edge_index.shape[1]
    d1 = W1.shape[1]
    d2 = W2.shape[1]
    assert n % BLK == 0
    grid = (n // BLK,)

    f32 = jnp.float32
    n_pad = ((n + 127) // 128) * 128
    ei3 = edge_index.astype(jnp.int32).reshape(2, e // CHUNK, CHUNK)
    degp = _make_deg_kernel(n_pad, e)(ei3)
    g1, dinv = pl.pallas_call(
        _lin1_body,
        grid=grid,
        in_specs=[_rows((n, d_in)), _full((d_in, d1)), _rows((NC, n_pad, 16))],
        out_specs=(_rows((n, d1)), _rows((n, 1))),
        out_shape=(
            jax.ShapeDtypeStruct((n, d1), f32),
            jax.ShapeDtypeStruct((n, 1), f32),
        ),
    )(x, W1, degp)
    s1p = _make_scatter_kernel(n, n_pad, e, d1)(g1, ei3)
    g2 = pl.pallas_call(
        _mid_body,
        grid=grid,
        in_specs=[_rows((NC, n_pad, d1)), _rows((n, d1)), _rows((n, 1)),
                  _full((1, d1)), _full((d1, d2))],
        out_specs=_rows((n, d2)),
        out_shape=jax.ShapeDtypeStruct((n, d2), f32),
    )(s1p, g1, dinv, b1.reshape(1, d1), W2)
    s2p = _make_scatter_kernel(n, n_pad, e, d2)(g2, ei3)
    out = pl.pallas_call(
        _out_body,
        grid=grid,
        in_specs=[_rows((NC, n_pad, d2)), _rows((n, d2)), _rows((n, 1)),
                  _full((1, d2)), _full((d2, 2)), _full((1, 2))],
        out_specs=_rows((n, 2)),
        out_shape=jax.ShapeDtypeStruct((n, 2), f32),
    )(s2p, g2, dinv, b2.reshape(1, d2), Wfc, bfc.reshape(1, 2))
    return out


# speedup vs baseline: 54.6291x; 1.0031x over previous
import functools

import jax
import jax.numpy as jnp
from jax import lax
from jax.experimental import pallas as pl
from jax.experimental.pallas import tpu as pltpu
from jax.experimental.pallas import tpu_sc as plsc

NC = 2
NS = 16
NW = NC * NS
CHUNK = 80
NBUF = 5


def _mesh():
    return plsc.VectorSubcoreMesh(
        core_axis_name="c", subcore_axis_name="s", num_cores=NC, num_subcores=NS
    )


_SC_PARAMS = pltpu.CompilerParams(use_tc_tiling_on_sc=False)


def _zero_shared(zbuf, acc, rows_per_tile, d, sid):
    def zstore(r, _):
        for j in range(d // 16):
            zbuf[r, pl.ds(j * 16, 16)] = jnp.zeros((16,), jnp.float32)
        return 0

    lax.fori_loop(0, rows_per_tile, zstore, 0)
    pltpu.sync_copy(zbuf, acc.at[pl.ds(sid * rows_per_tile, rows_per_tile)])


def _copy_small(src_all, buf, off, n):
    for j in range(n // 16):
        buf[pl.ds(j * 16, 16)] = src_all[pl.ds(off + j * 16, 16)]


def _make_deg_kernel(n_pad, e):
    e_per_w = e // NW
    assert e_per_w % (CHUNK * NBUF) == 0
    nch = e_per_w // CHUNK
    rows_per_tile = n_pad // NS

    @functools.partial(
        pl.kernel,
        out_type=jax.ShapeDtypeStruct((NC, n_pad, 16), jnp.float32),
        mesh=_mesh(),
        scratch_types=[
            pltpu.VMEM((nch, CHUNK), jnp.int32),
            pltpu.VMEM((CHUNK, 16), jnp.float32),
            pltpu.VMEM((rows_per_tile, 16), jnp.float32),
            pltpu.VMEM_SHARED((n_pad, 16), jnp.float32),
            [pltpu.SemaphoreType.DMA for _ in range(NBUF)],
        ],
        compiler_params=_SC_PARAMS,
    )
    def deg_kernel(ei_hbm, out_hbm, dst_all, ones, zbuf, acc, ssem):
        cid = lax.axis_index("c")
        sid = lax.axis_index("s")
        wid = cid * NS + sid
        base = wid * nch

        def fill_ones(r, _):
            ones[r, pl.ds(0, 16)] = jnp.ones((16,), jnp.float32)
            return 0

        lax.fori_loop(0, CHUNK, fill_ones, 0)
        _zero_shared(zbuf, acc, rows_per_tile, 16, sid)
        pltpu.sync_copy(ei_hbm.at[1, pl.ds(base, nch)], dst_all)
        plsc.subcore_barrier()

        def outer(o, _):
            for b in range(NBUF):
                i = o * NBUF + b

                @pl.when(o > 0)
                def _wait_prev_scatter():
                    pltpu.make_async_copy(
                        ones, acc.at[dst_all.at[i - NBUF]], ssem[b]
                    ).wait()

                pltpu.async_copy(ones, acc.at[dst_all.at[i]], ssem[b], add=True)
            return 0

        lax.fori_loop(0, nch // NBUF, outer, 0)
        for b in range(NBUF):
            i = nch - NBUF + b
            pltpu.make_async_copy(ones, acc.at[dst_all.at[i]], ssem[b]).wait()
        plsc.subcore_barrier()
        pltpu.sync_copy(
            acc.at[pl.ds(sid * rows_per_tile, rows_per_tile)],
            out_hbm.at[cid, pl.ds(sid * rows_per_tile, rows_per_tile)],
        )

    return deg_kernel


def _make_scatter_kernel(n, n_pad, e, d):
    e_per_w = e // NW
    assert e_per_w % (CHUNK * NBUF) == 0
    rows_per_tile = n_pad // NS

    nch = e_per_w // CHUNK
    scratch_types = [
        pltpu.VMEM((nch, CHUNK), jnp.int32),
        pltpu.VMEM((nch, CHUNK), jnp.int32),
        [pltpu.VMEM((CHUNK, d), jnp.float32) for _ in range(NBUF)],
        pltpu.VMEM((rows_per_tile, d), jnp.float32),
        pltpu.VMEM_SHARED((n_pad, d), jnp.float32),
        [pltpu.SemaphoreType.DMA for _ in range(NBUF)],
        [pltpu.SemaphoreType.DMA for _ in range(NBUF)],
    ]

    @functools.partial(
        pl.kernel,
        out_type=jax.ShapeDtypeStruct((NC, n_pad, d), jnp.float32),
        mesh=_mesh(),
        scratch_types=scratch_types,
        compiler_params=_SC_PARAMS,
    )
    def scatter_kernel(
        g_hbm, ei_hbm, out_hbm,
        src_all, dst_all, rows, zbuf, acc, gsem, ssem,
    ):
        cid = lax.axis_index("c")
        sid = lax.axis_index("s")
        wid = cid * NS + sid
        base = wid * nch

        _zero_shared(zbuf, acc, rows_per_tile, d, sid)
        pltpu.sync_copy(ei_hbm.at[0, pl.ds(base, nch)], src_all)
        pltpu.sync_copy(ei_hbm.at[1, pl.ds(base, nch)], dst_all)
        plsc.subcore_barrier()

        def outer(o, _):
            for b in range(NBUF):
                i = o * NBUF + b

                @pl.when(o > 0)
                def _wait_prev_scatter():
                    pltpu.make_async_copy(
                        rows[b], acc.at[dst_all.at[i - NBUF]], ssem[b]
                    ).wait()

                pltpu.async_copy(g_hbm.at[src_all.at[i]], rows[b], gsem[b])
            for b in range(NBUF):
                i = o * NBUF + b
                pltpu.make_async_copy(
                    g_hbm.at[src_all.at[i]], rows[b], gsem[b]
                ).wait()
                pltpu.async_copy(rows[b], acc.at[dst_all.at[i]], ssem[b], add=True)
            return 0

        lax.fori_loop(0, nch // NBUF, outer, 0)
        for b in range(NBUF):
            i = nch - NBUF + b
            pltpu.make_async_copy(rows[b], acc.at[dst_all.at[i]], ssem[b]).wait()
        plsc.subcore_barrier()
        pltpu.sync_copy(
            acc.at[pl.ds(sid * rows_per_tile, rows_per_tile)],
            out_hbm.at[cid, pl.ds(sid * rows_per_tile, rows_per_tile)],
        )

    return scatter_kernel


BLK = 2000


def _mm1_body(x_ref, w_ref, h1_ref):
    h1_ref[...] = jnp.dot(x_ref[...], w_ref[...], preferred_element_type=jnp.float32)


def _scale1_body(h1_ref, degp_ref, g1_ref, dinv_ref):
    dp = degp_ref[...]
    deg = 1.0 + dp[0, :, 0:1] + dp[1, :, 0:1]
    dinv = lax.rsqrt(deg)
    dinv_ref[...] = dinv
    g1_ref[...] = h1_ref[...] * dinv


def _mid_body(sp_ref, g_ref, dinv_ref, b_ref, w_ref, o_ref):
    p = sp_ref[...]
    s = g_ref[...] + p[0] + p[1]
    dinv = dinv_ref[...]
    a = jnp.maximum(s * dinv + b_ref[...], 0.0)
    o_ref[...] = jnp.dot(a, w_ref[...], preferred_element_type=jnp.float32) * dinv


def _out_body(sp_ref, g_ref, dinv_ref, b_ref, wfc_ref, bfc_ref, o_ref):
    p = sp_ref[...]
    s = g_ref[...] + p[0] + p[1]
    a = jnp.maximum(s * dinv_ref[...] + b_ref[...], 0.0)
    h = jnp.dot(a, wfc_ref[...], preferred_element_type=jnp.float32) + bfc_ref[...]
    m = jnp.max(h, axis=1, keepdims=True)
    lse = m + jnp.log(jnp.sum(jnp.exp(h - m), axis=1, keepdims=True))
    o_ref[...] = h - lse


def _rows(shape):
    if len(shape) == 2:
        return pl.BlockSpec((BLK, shape[1]), lambda i: (i, 0))
    return pl.BlockSpec((shape[0], BLK, shape[2]), lambda i: (0, i, 0))


def _full(shape):
    return pl.BlockSpec(shape, lambda i: tuple(0 for _ in shape))


def kernel(x, edge_index, W1, b1, W2, b2, Wfc, bfc):
    n, d_in = x.shape
    e = edge_index.shape[1]
    d1 = W1.shape[1]
    d2 = W2.shape[1]
    assert n % BLK == 0
    grid = (n // BLK,)

    f32 = jnp.float32
    n_pad = ((n + 127) // 128) * 128
    ei3 = edge_index.astype(jnp.int32).reshape(2, e // CHUNK, CHUNK)
    degp = _make_deg_kernel(n_pad, e)(ei3)
    h1 = pl.pallas_call(
        _mm1_body,
        grid=grid,
        in_specs=[_rows((n, d_in)), _full((d_in, d1))],
        out_specs=_rows((n, d1)),
        out_shape=jax.ShapeDtypeStruct((n, d1), f32),
    )(x, W1)
    g1, dinv = pl.pallas_call(
        _scale1_body,
        grid=grid,
        in_specs=[_rows((n, d1)), _rows((NC, n_pad, 16))],
        out_specs=(_rows((n, d1)), _rows((n, 1))),
        out_shape=(
            jax.ShapeDtypeStruct((n, d1), f32),
            jax.ShapeDtypeStruct((n, 1), f32),
        ),
    )(h1, degp)
    s1p = _make_scatter_kernel(n, n_pad, e, d1)(g1, ei3)
    g2 = pl.pallas_call(
        _mid_body,
        grid=grid,
        in_specs=[_rows((NC, n_pad, d1)), _rows((n, d1)), _rows((n, 1)),
                  _full((1, d1)), _full((d1, d2))],
        out_specs=_rows((n, d2)),
        out_shape=jax.ShapeDtypeStruct((n, d2), f32),
    )(s1p, g1, dinv, b1.reshape(1, d1), W2)
    s2p = _make_scatter_kernel(n, n_pad, e, d2)(g2, ei3)
    out = pl.pallas_call(
        _out_body,
        grid=grid,
        in_specs=[_rows((NC, n_pad, d2)), _rows((n, d2)), _rows((n, 1)),
                  _full((1, d2)), _full((d2, 2)), _full((1, 2))],
        out_specs=_rows((n, 2)),
        out_shape=jax.ShapeDtypeStruct((n, 2), f32),
    )(s2p, g2, dinv, b2.reshape(1, d2), Wfc, bfc.reshape(1, 2))
    return out


# speedup vs baseline: 56.2428x; 1.0295x over previous
import functools

import jax
import jax.numpy as jnp
from jax import lax
from jax.experimental import pallas as pl
from jax.experimental.pallas import tpu as pltpu
from jax.experimental.pallas import tpu_sc as plsc

NC = 2
NS = 16
NW = NC * NS
CHUNK = 128
NBUF = 6


def _mesh():
    return plsc.VectorSubcoreMesh(
        core_axis_name="c", subcore_axis_name="s", num_cores=NC, num_subcores=NS
    )


_SC_PARAMS = pltpu.CompilerParams(use_tc_tiling_on_sc=False)


def _zero_shared(zbuf, acc, rows_per_tile, d, sid):
    def zstore(r, _):
        for j in range(d // 16):
            zbuf[r, pl.ds(j * 16, 16)] = jnp.zeros((16,), jnp.float32)
        return 0

    lax.fori_loop(0, rows_per_tile, zstore, 0)
    pltpu.sync_copy(zbuf, acc.at[pl.ds(sid * rows_per_tile, rows_per_tile)])


def _copy_small(src_all, buf, off, n):
    for j in range(n // 16):
        buf[pl.ds(j * 16, 16)] = src_all[pl.ds(off + j * 16, 16)]


def _make_deg_kernel(n_pad, e):
    e_rows = e // CHUNK
    nch = e_rows // NW
    n_extra = e_rows - nch * NW
    assert nch % NBUF == 0
    rows_per_tile = n_pad // NS

    @functools.partial(
        pl.kernel,
        out_type=jax.ShapeDtypeStruct((NC, n_pad, 16), jnp.float32),
        mesh=_mesh(),
        scratch_types=[
            pltpu.VMEM((nch + 1, CHUNK), jnp.int32),
            pltpu.VMEM((CHUNK, 16), jnp.float32),
            pltpu.VMEM((rows_per_tile, 16), jnp.float32),
            pltpu.VMEM_SHARED((n_pad, 16), jnp.float32),
            [pltpu.SemaphoreType.DMA for _ in range(NBUF)],
        ],
        compiler_params=_SC_PARAMS,
    )
    def deg_kernel(ei_hbm, out_hbm, dst_all, ones, zbuf, acc, ssem):
        cid = lax.axis_index("c")
        sid = lax.axis_index("s")
        wid = cid * NS + sid
        base = wid * nch
        xrow = nch * NW + jnp.minimum(wid, n_extra - 1)

        def fill_ones(r, _):
            ones[r, pl.ds(0, 16)] = jnp.ones((16,), jnp.float32)
            return 0

        lax.fori_loop(0, CHUNK, fill_ones, 0)
        _zero_shared(zbuf, acc, rows_per_tile, 16, sid)
        pltpu.sync_copy(ei_hbm.at[1, pl.ds(base, nch)], dst_all.at[pl.ds(0, nch)])
        pltpu.sync_copy(ei_hbm.at[1, pl.ds(xrow, 1)], dst_all.at[pl.ds(nch, 1)])
        plsc.subcore_barrier()

        def outer(o, _):
            for b in range(NBUF):
                i = o * NBUF + b

                @pl.when(o > 0)
                def _wait_prev_scatter():
                    pltpu.make_async_copy(
                        ones, acc.at[dst_all.at[i - NBUF]], ssem[b]
                    ).wait()

                pltpu.async_copy(ones, acc.at[dst_all.at[i]], ssem[b], add=True)
            return 0

        lax.fori_loop(0, nch // NBUF, outer, 0)
        for b in range(NBUF):
            i = nch - NBUF + b
            pltpu.make_async_copy(ones, acc.at[dst_all.at[i]], ssem[b]).wait()

        @pl.when(wid < n_extra)
        def _extra():
            pltpu.sync_copy(ones, acc.at[dst_all.at[nch]], add=True)

        plsc.subcore_barrier()
        pltpu.sync_copy(
            acc.at[pl.ds(sid * rows_per_tile, rows_per_tile)],
            out_hbm.at[cid, pl.ds(sid * rows_per_tile, rows_per_tile)],
        )

    return deg_kernel


def _make_scatter_kernel(n, n_pad, e, d):
    e_rows = e // CHUNK
    nch = e_rows // NW
    n_extra = e_rows - nch * NW
    assert nch % NBUF == 0
    rows_per_tile = n_pad // NS

    scratch_types = [
        pltpu.VMEM((nch + 1, CHUNK), jnp.int32),
        pltpu.VMEM((nch + 1, CHUNK), jnp.int32),
        [pltpu.VMEM((CHUNK, d), jnp.float32) for _ in range(NBUF)],
        pltpu.VMEM((rows_per_tile, d), jnp.float32),
        pltpu.VMEM_SHARED((n_pad, d), jnp.float32),
        [pltpu.SemaphoreType.DMA for _ in range(NBUF)],
        [pltpu.SemaphoreType.DMA for _ in range(NBUF)],
    ]

    @functools.partial(
        pl.kernel,
        out_type=jax.ShapeDtypeStruct((NC, n_pad, d), jnp.float32),
        mesh=_mesh(),
        scratch_types=scratch_types,
        compiler_params=_SC_PARAMS,
    )
    def scatter_kernel(
        g_hbm, ei_hbm, out_hbm,
        src_all, dst_all, rows, zbuf, acc, gsem, ssem,
    ):
        cid = lax.axis_index("c")
        sid = lax.axis_index("s")
        wid = cid * NS + sid
        base = wid * nch
        xrow = nch * NW + jnp.minimum(wid, n_extra - 1)

        _zero_shared(zbuf, acc, rows_per_tile, d, sid)
        pltpu.sync_copy(ei_hbm.at[0, pl.ds(base, nch)], src_all.at[pl.ds(0, nch)])
        pltpu.sync_copy(ei_hbm.at[1, pl.ds(base, nch)], dst_all.at[pl.ds(0, nch)])
        pltpu.sync_copy(ei_hbm.at[0, pl.ds(xrow, 1)], src_all.at[pl.ds(nch, 1)])
        pltpu.sync_copy(ei_hbm.at[1, pl.ds(xrow, 1)], dst_all.at[pl.ds(nch, 1)])
        plsc.subcore_barrier()

        def outer(o, _):
            for b in range(NBUF):
                i = o * NBUF + b

                @pl.when(o > 0)
                def _wait_prev_scatter():
                    pltpu.make_async_copy(
                        rows[b], acc.at[dst_all.at[i - NBUF]], ssem[b]
                    ).wait()

                pltpu.async_copy(g_hbm.at[src_all.at[i]], rows[b], gsem[b])
            for b in range(NBUF):
                i = o * NBUF + b
                pltpu.make_async_copy(
                    g_hbm.at[src_all.at[i]], rows[b], gsem[b]
                ).wait()
                pltpu.async_copy(rows[b], acc.at[dst_all.at[i]], ssem[b], add=True)
            return 0

        lax.fori_loop(0, nch // NBUF, outer, 0)
        for b in range(NBUF):
            i = nch - NBUF + b
            pltpu.make_async_copy(rows[b], acc.at[dst_all.at[i]], ssem[b]).wait()

        @pl.when(wid < n_extra)
        def _extra():
            pltpu.sync_copy(g_hbm.at[src_all.at[nch]], rows[0])
            pltpu.sync_copy(rows[0], acc.at[dst_all.at[nch]], add=True)

        plsc.subcore_barrier()
        pltpu.sync_copy(
            acc.at[pl.ds(sid * rows_per_tile, rows_per_tile)],
            out_hbm.at[cid, pl.ds(sid * rows_per_tile, rows_per_tile)],
        )

    return scatter_kernel


BLK = 2000


def _mm1_body(x_ref, w_ref, h1_ref):
    h1_ref[...] = jnp.dot(x_ref[...], w_ref[...], preferred_element_type=jnp.float32)


def _scale1_body(h1_ref, degp_ref, g1_ref, dinv_ref):
    dp = degp_ref[...]
    deg = 1.0 + dp[0, :, 0:1] + dp[1, :, 0:1]
    dinv = lax.rsqrt(deg)
    dinv_ref[...] = dinv
    g1_ref[...] = h1_ref[...] * dinv


def _mid_body(sp_ref, g_ref, dinv_ref, b_ref, w_ref, o_ref):
    p = sp_ref[...]
    s = g_ref[...] + p[0] + p[1]
    dinv = dinv_ref[...]
    a = jnp.maximum(s * dinv + b_ref[...], 0.0)
    o_ref[...] = jnp.dot(a, w_ref[...], preferred_element_type=jnp.float32) * dinv


def _out_body(sp_ref, g_ref, dinv_ref, b_ref, wfc_ref, bfc_ref, o_ref):
    p = sp_ref[...]
    s = g_ref[...] + p[0] + p[1]
    a = jnp.maximum(s * dinv_ref[...] + b_ref[...], 0.0)
    h = jnp.dot(a, wfc_ref[...], preferred_element_type=jnp.float32) + bfc_ref[...]
    m = jnp.max(h, axis=1, keepdims=True)
    lse = m + jnp.log(jnp.sum(jnp.exp(h - m), axis=1, keepdims=True))
    o_ref[...] = h - lse


def _rows(shape):
    if len(shape) == 2:
        return pl.BlockSpec((BLK, shape[1]), lambda i: (i, 0))
    return pl.BlockSpec((shape[0], BLK, shape[2]), lambda i: (0, i, 0))


def _full(shape):
    return pl.BlockSpec(shape, lambda i: tuple(0 for _ in shape))


def kernel(x, edge_index, W1, b1, W2, b2, Wfc, bfc):
    n, d_in = x.shape
    e = edge_index.shape[1]
    d1 = W1.shape[1]
    d2 = W2.shape[1]
    assert n % BLK == 0
    grid = (n // BLK,)

    f32 = jnp.float32
    n_pad = ((n + 127) // 128) * 128
    ei3 = edge_index.astype(jnp.int32).reshape(2, e // CHUNK, CHUNK)
    degp = _make_deg_kernel(n_pad, e)(ei3)
    h1 = pl.pallas_call(
        _mm1_body,
        grid=grid,
        in_specs=[_rows((n, d_in)), _full((d_in, d1))],
        out_specs=_rows((n, d1)),
        out_shape=jax.ShapeDtypeStruct((n, d1), f32),
    )(x, W1)
    g1, dinv = pl.pallas_call(
        _scale1_body,
        grid=grid,
        in_specs=[_rows((n, d1)), _rows((NC, n_pad, 16))],
        out_specs=(_rows((n, d1)), _rows((n, 1))),
        out_shape=(
            jax.ShapeDtypeStruct((n, d1), f32),
            jax.ShapeDtypeStruct((n, 1), f32),
        ),
    )(h1, degp)
    s1p = _make_scatter_kernel(n, n_pad, e, d1)(g1, ei3)
    g2 = pl.pallas_call(
        _mid_body,
        grid=grid,
        in_specs=[_rows((NC, n_pad, d1)), _rows((n, d1)), _rows((n, 1)),
                  _full((1, d1)), _full((d1, d2))],
        out_specs=_rows((n, d2)),
        out_shape=jax.ShapeDtypeStruct((n, d2), f32),
    )(s1p, g1, dinv, b1.reshape(1, d1), W2)
    s2p = _make_scatter_kernel(n, n_pad, e, d2)(g2, ei3)
    out = pl.pallas_call(
        _out_body,
        grid=grid,
        in_specs=[_rows((NC, n_pad, d2)), _rows((n, d2)), _rows((n, 1)),
                  _full((1, d2)), _full((d2, 2)), _full((1, 2))],
        out_specs=_rows((n, 2)),
        out_shape=jax.ShapeDtypeStruct((n, 2), f32),
    )(s2p, g2, dinv, b2.reshape(1, d2), Wfc, bfc.reshape(1, 2))
    return out


# speedup vs baseline: 57.6714x; 1.0254x over previous
import functools

import jax
import jax.numpy as jnp
from jax import lax
from jax.experimental import pallas as pl
from jax.experimental.pallas import tpu as pltpu
from jax.experimental.pallas import tpu_sc as plsc

NC = 2
NS = 16
NW = NC * NS
CHUNK = 128
NBUF = 13


def _mesh():
    return plsc.VectorSubcoreMesh(
        core_axis_name="c", subcore_axis_name="s", num_cores=NC, num_subcores=NS
    )


_SC_PARAMS = pltpu.CompilerParams(use_tc_tiling_on_sc=False)


def _zero_shared(zbuf, acc, rows_per_tile, d, sid):
    def zstore(r, _):
        for j in range(d // 16):
            zbuf[r, pl.ds(j * 16, 16)] = jnp.zeros((16,), jnp.float32)
        return 0

    lax.fori_loop(0, rows_per_tile, zstore, 0)
    pltpu.sync_copy(zbuf, acc.at[pl.ds(sid * rows_per_tile, rows_per_tile)])


def _copy_small(src_all, buf, off, n):
    for j in range(n // 16):
        buf[pl.ds(j * 16, 16)] = src_all[pl.ds(off + j * 16, 16)]


def _make_deg_kernel(n_pad, e):
    e_rows = e // CHUNK
    nch = e_rows // NW
    n_extra = e_rows - nch * NW
    assert nch % NBUF == 0
    rows_per_tile = n_pad // NS

    @functools.partial(
        pl.kernel,
        out_type=jax.ShapeDtypeStruct((NC, n_pad, 16), jnp.float32),
        mesh=_mesh(),
        scratch_types=[
            pltpu.VMEM((nch + 1, CHUNK), jnp.int32),
            pltpu.VMEM((CHUNK, 16), jnp.float32),
            pltpu.VMEM((rows_per_tile, 16), jnp.float32),
            pltpu.VMEM_SHARED((n_pad, 16), jnp.float32),
            [pltpu.SemaphoreType.DMA for _ in range(NBUF)],
        ],
        compiler_params=_SC_PARAMS,
    )
    def deg_kernel(ei_hbm, out_hbm, dst_all, ones, zbuf, acc, ssem):
        cid = lax.axis_index("c")
        sid = lax.axis_index("s")
        wid = cid * NS + sid
        base = wid * nch
        xrow = nch * NW + jnp.minimum(wid, n_extra - 1)

        def fill_ones(r, _):
            ones[r, pl.ds(0, 16)] = jnp.ones((16,), jnp.float32)
            return 0

        lax.fori_loop(0, CHUNK, fill_ones, 0)
        _zero_shared(zbuf, acc, rows_per_tile, 16, sid)
        pltpu.sync_copy(ei_hbm.at[1, pl.ds(base, nch)], dst_all.at[pl.ds(0, nch)])
        pltpu.sync_copy(ei_hbm.at[1, pl.ds(xrow, 1)], dst_all.at[pl.ds(nch, 1)])
        plsc.subcore_barrier()

        def outer(o, _):
            for b in range(NBUF):
                i = o * NBUF + b

                @pl.when(o > 0)
                def _wait_prev_scatter():
                    pltpu.make_async_copy(
                        ones, acc.at[dst_all.at[i - NBUF]], ssem[b]
                    ).wait()

                pltpu.async_copy(ones, acc.at[dst_all.at[i]], ssem[b], add=True)
            return 0

        lax.fori_loop(0, nch // NBUF, outer, 0)
        for b in range(NBUF):
            i = nch - NBUF + b
            pltpu.make_async_copy(ones, acc.at[dst_all.at[i]], ssem[b]).wait()

        @pl.when(wid < n_extra)
        def _extra():
            pltpu.sync_copy(ones, acc.at[dst_all.at[nch]], add=True)

        plsc.subcore_barrier()
        pltpu.sync_copy(
            acc.at[pl.ds(sid * rows_per_tile, rows_per_tile)],
            out_hbm.at[cid, pl.ds(sid * rows_per_tile, rows_per_tile)],
        )

    return deg_kernel


def _make_scatter_kernel(n, n_pad, e, d):
    e_rows = e // CHUNK
    nch = e_rows // NW
    n_extra = e_rows - nch * NW
    assert nch % NBUF == 0
    rows_per_tile = n_pad // NS

    scratch_types = [
        pltpu.VMEM((nch + 1, CHUNK), jnp.int32),
        pltpu.VMEM((nch + 1, CHUNK), jnp.int32),
        [pltpu.VMEM((CHUNK, d), jnp.float32) for _ in range(NBUF)],
        pltpu.VMEM((rows_per_tile, d), jnp.float32),
        pltpu.VMEM_SHARED((n_pad, d), jnp.float32),
        [pltpu.SemaphoreType.DMA for _ in range(NBUF)],
        [pltpu.SemaphoreType.DMA for _ in range(NBUF)],
    ]

    @functools.partial(
        pl.kernel,
        out_type=jax.ShapeDtypeStruct((NC, n_pad, d), jnp.float32),
        mesh=_mesh(),
        scratch_types=scratch_types,
        compiler_params=_SC_PARAMS,
    )
    def scatter_kernel(
        g_hbm, ei_hbm, out_hbm,
        src_all, dst_all, rows, zbuf, acc, gsem, ssem,
    ):
        cid = lax.axis_index("c")
        sid = lax.axis_index("s")
        wid = cid * NS + sid
        base = wid * nch
        xrow = nch * NW + jnp.minimum(wid, n_extra - 1)

        _zero_shared(zbuf, acc, rows_per_tile, d, sid)
        pltpu.sync_copy(ei_hbm.at[0, pl.ds(base, nch)], src_all.at[pl.ds(0, nch)])
        pltpu.sync_copy(ei_hbm.at[1, pl.ds(base, nch)], dst_all.at[pl.ds(0, nch)])
        pltpu.sync_copy(ei_hbm.at[0, pl.ds(xrow, 1)], src_all.at[pl.ds(nch, 1)])
        pltpu.sync_copy(ei_hbm.at[1, pl.ds(xrow, 1)], dst_all.at[pl.ds(nch, 1)])
        plsc.subcore_barrier()

        def outer(o, _):
            for b in range(NBUF):
                i = o * NBUF + b

                @pl.when(o > 0)
                def _wait_prev_scatter():
                    pltpu.make_async_copy(
                        rows[b], acc.at[dst_all.at[i - NBUF]], ssem[b]
                    ).wait()

                pltpu.async_copy(g_hbm.at[src_all.at[i]], rows[b], gsem[b])
            for b in range(NBUF):
                i = o * NBUF + b
                pltpu.make_async_copy(
                    g_hbm.at[src_all.at[i]], rows[b], gsem[b]
                ).wait()
                pltpu.async_copy(rows[b], acc.at[dst_all.at[i]], ssem[b], add=True)
            return 0

        lax.fori_loop(0, nch // NBUF, outer, 0)
        for b in range(NBUF):
            i = nch - NBUF + b
            pltpu.make_async_copy(rows[b], acc.at[dst_all.at[i]], ssem[b]).wait()

        @pl.when(wid < n_extra)
        def _extra():
            pltpu.sync_copy(g_hbm.at[src_all.at[nch]], rows[0])
            pltpu.sync_copy(rows[0], acc.at[dst_all.at[nch]], add=True)

        plsc.subcore_barrier()
        pltpu.sync_copy(
            acc.at[pl.ds(sid * rows_per_tile, rows_per_tile)],
            out_hbm.at[cid, pl.ds(sid * rows_per_tile, rows_per_tile)],
        )

    return scatter_kernel


BLK = 2000


def _mm1_body(x_ref, w_ref, h1_ref):
    h1_ref[...] = jnp.dot(x_ref[...], w_ref[...], preferred_element_type=jnp.float32)


def _scale1_body(h1_ref, degp_ref, g1_ref, dinv_ref):
    dp = degp_ref[...]
    deg = 1.0 + dp[0, :, 0:1] + dp[1, :, 0:1]
    dinv = lax.rsqrt(deg)
    dinv_ref[...] = dinv
    g1_ref[...] = h1_ref[...] * dinv


def _mid_body(sp_ref, g_ref, dinv_ref, b_ref, w_ref, o_ref):
    p = sp_ref[...]
    s = g_ref[...] + p[0] + p[1]
    dinv = dinv_ref[...]
    a = jnp.maximum(s * dinv + b_ref[...], 0.0)
    o_ref[...] = jnp.dot(a, w_ref[...], preferred_element_type=jnp.float32) * dinv


def _out_body(sp_ref, g_ref, dinv_ref, b_ref, wfc_ref, bfc_ref, o_ref):
    p = sp_ref[...]
    s = g_ref[...] + p[0] + p[1]
    a = jnp.maximum(s * dinv_ref[...] + b_ref[...], 0.0)
    h = jnp.dot(a, wfc_ref[...], preferred_element_type=jnp.float32) + bfc_ref[...]
    m = jnp.max(h, axis=1, keepdims=True)
    lse = m + jnp.log(jnp.sum(jnp.exp(h - m), axis=1, keepdims=True))
    o_ref[...] = h - lse


def _rows(shape):
    if len(shape) == 2:
        return pl.BlockSpec((BLK, shape[1]), lambda i: (i, 0))
    return pl.BlockSpec((shape[0], BLK, shape[2]), lambda i: (0, i, 0))


def _full(shape):
    return pl.BlockSpec(shape, lambda i: tuple(0 for _ in shape))


def kernel(x, edge_index, W1, b1, W2, b2, Wfc, bfc):
    n, d_in = x.shape
    e = edge_index.shape[1]
    d1 = W1.shape[1]
    d2 = W2.shape[1]
    assert n % BLK == 0
    grid = (n // BLK,)

    f32 = jnp.float32
    n_pad = ((n + 127) // 128) * 128
    ei3 = edge_index.astype(jnp.int32).reshape(2, e // CHUNK, CHUNK)
    degp = _make_deg_kernel(n_pad, e)(ei3)
    h1 = pl.pallas_call(
        _mm1_body,
        grid=grid,
        in_specs=[_rows((n, d_in)), _full((d_in, d1))],
        out_specs=_rows((n, d1)),
        out_shape=jax.ShapeDtypeStruct((n, d1), f32),
    )(x, W1)
    g1, dinv = pl.pallas_call(
        _scale1_body,
        grid=grid,
        in_specs=[_rows((n, d1)), _rows((NC, n_pad, 16))],
        out_specs=(_rows((n, d1)), _rows((n, 1))),
        out_shape=(
            jax.ShapeDtypeStruct((n, d1), f32),
            jax.ShapeDtypeStruct((n, 1), f32),
        ),
    )(h1, degp)
    s1p = _make_scatter_kernel(n, n_pad, e, d1)(g1, ei3)
    g2 = pl.pallas_call(
        _mid_body,
        grid=grid,
        in_specs=[_rows((NC, n_pad, d1)), _rows((n, d1)), _rows((n, 1)),
                  _full((1, d1)), _full((d1, d2))],
        out_specs=_rows((n, d2)),
        out_shape=jax.ShapeDtypeStruct((n, d2), f32),
    )(s1p, g1, dinv, b1.reshape(1, d1), W2)
    s2p = _make_scatter_kernel(n, n_pad, e, d2)(g2, ei3)
    out = pl.pallas_call(
        _out_body,
        grid=grid,
        in_specs=[_rows((NC, n_pad, d2)), _rows((n, d2)), _rows((n, 1)),
                  _full((1, d2)), _full((d2, 2)), _full((1, 2))],
        out_specs=_rows((n, 2)),
        out_shape=jax.ShapeDtypeStruct((n, 2), f32),
    )(s2p, g2, dinv, b2.reshape(1, d2), Wfc, bfc.reshape(1, 2))
    return out


# speedup vs baseline: 63.7966x; 1.1062x over previous
import functools

import jax
import jax.numpy as jnp
from jax import lax
from jax.experimental import pallas as pl
from jax.experimental.pallas import tpu as pltpu
from jax.experimental.pallas import tpu_sc as plsc

NC = 2
NS = 16
NW = NC * NS
CHUNK = 128
NBUF = 13


def _mesh():
    return plsc.VectorSubcoreMesh(
        core_axis_name="c", subcore_axis_name="s", num_cores=NC, num_subcores=NS
    )


_SC_PARAMS = pltpu.CompilerParams(use_tc_tiling_on_sc=False)


def _zero_shared(zbuf, acc, rows_per_tile, d, sid):
    def zstore(r, _):
        for j in range(d // 16):
            zbuf[r, pl.ds(j * 16, 16)] = jnp.zeros((16,), jnp.float32)
        return 0

    lax.fori_loop(0, rows_per_tile, zstore, 0)
    pltpu.sync_copy(zbuf, acc.at[pl.ds(sid * rows_per_tile, rows_per_tile)])


def _copy_small(src_all, buf, off, n):
    for j in range(n // 16):
        buf[pl.ds(j * 16, 16)] = src_all[pl.ds(off + j * 16, 16)]


def _make_deg_kernel(n_pad, e):
    e_rows = e // CHUNK
    nch = e_rows // NW
    n_extra = e_rows - nch * NW
    assert nch % NBUF == 0
    rows_per_tile = n_pad // NS

    @functools.partial(
        pl.kernel,
        out_type=jax.ShapeDtypeStruct((NC, n_pad * 16 // 128, 128), jnp.float32),
        mesh=_mesh(),
        scratch_types=[
            pltpu.VMEM((nch + 1, CHUNK), jnp.int32),
            pltpu.VMEM((CHUNK, 16), jnp.float32),
            pltpu.VMEM((rows_per_tile, 16), jnp.float32),
            pltpu.VMEM_SHARED((n_pad, 16), jnp.float32),
            [pltpu.SemaphoreType.DMA for _ in range(NBUF)],
        ],
        compiler_params=_SC_PARAMS,
    )
    def deg_kernel(ei_hbm, out_hbm, dst_all, ones, zbuf, acc, ssem):
        cid = lax.axis_index("c")
        sid = lax.axis_index("s")
        wid = cid * NS + sid
        base = wid * nch
        xrow = nch * NW + jnp.minimum(wid, n_extra - 1)

        def fill_ones(r, _):
            ones[r, pl.ds(0, 16)] = jnp.ones((16,), jnp.float32)
            return 0

        lax.fori_loop(0, CHUNK, fill_ones, 0)
        _zero_shared(zbuf, acc, rows_per_tile, 16, sid)
        pltpu.sync_copy(ei_hbm.at[1, pl.ds(base, nch)], dst_all.at[pl.ds(0, nch)])
        pltpu.sync_copy(ei_hbm.at[1, pl.ds(xrow, 1)], dst_all.at[pl.ds(nch, 1)])
        plsc.subcore_barrier()

        def outer(o, _):
            for b in range(NBUF):
                i = o * NBUF + b

                @pl.when(o > 0)
                def _wait_prev_scatter():
                    pltpu.make_async_copy(
                        ones, acc.at[dst_all.at[i - NBUF]], ssem[b]
                    ).wait()

                pltpu.async_copy(ones, acc.at[dst_all.at[i]], ssem[b], add=True)
            return 0

        lax.fori_loop(0, nch // NBUF, outer, 0)
        for b in range(NBUF):
            i = nch - NBUF + b
            pltpu.make_async_copy(ones, acc.at[dst_all.at[i]], ssem[b]).wait()

        @pl.when(wid < n_extra)
        def _extra():
            pltpu.sync_copy(ones, acc.at[dst_all.at[nch]], add=True)

        plsc.subcore_barrier()
        tpq = NS // (128 // 16)
        q = sid // tpq
        r0 = (sid % tpq) * rows_per_tile
        pltpu.sync_copy(
            acc.at[pl.ds(sid * rows_per_tile, rows_per_tile)],
            out_hbm.at[cid, pl.ds(r0, rows_per_tile), pl.ds(q * 16, 16)],
        )

    return deg_kernel


def _make_scatter_kernel(n, n_pad, e, d):
    e_rows = e // CHUNK
    nch = e_rows // NW
    n_extra = e_rows - nch * NW
    assert nch % NBUF == 0
    rows_per_tile = n_pad // NS

    scratch_types = [
        pltpu.VMEM((nch + 1, CHUNK), jnp.int32),
        pltpu.VMEM((nch + 1, CHUNK), jnp.int32),
        [pltpu.VMEM((CHUNK, d), jnp.float32) for _ in range(NBUF)],
        pltpu.VMEM((rows_per_tile, d), jnp.float32),
        pltpu.VMEM_SHARED((n_pad, d), jnp.float32),
        [pltpu.SemaphoreType.DMA for _ in range(NBUF)],
        [pltpu.SemaphoreType.DMA for _ in range(NBUF)],
    ]

    @functools.partial(
        pl.kernel,
        out_type=jax.ShapeDtypeStruct((NC, n_pad * d // 128, 128), jnp.float32),
        mesh=_mesh(),
        scratch_types=scratch_types,
        compiler_params=_SC_PARAMS,
    )
    def scatter_kernel(
        g_hbm, ei_hbm, out_hbm,
        src_all, dst_all, rows, zbuf, acc, gsem, ssem,
    ):
        cid = lax.axis_index("c")
        sid = lax.axis_index("s")
        wid = cid * NS + sid
        base = wid * nch
        xrow = nch * NW + jnp.minimum(wid, n_extra - 1)

        _zero_shared(zbuf, acc, rows_per_tile, d, sid)
        pltpu.sync_copy(ei_hbm.at[0, pl.ds(base, nch)], src_all.at[pl.ds(0, nch)])
        pltpu.sync_copy(ei_hbm.at[1, pl.ds(base, nch)], dst_all.at[pl.ds(0, nch)])
        pltpu.sync_copy(ei_hbm.at[0, pl.ds(xrow, 1)], src_all.at[pl.ds(nch, 1)])
        pltpu.sync_copy(ei_hbm.at[1, pl.ds(xrow, 1)], dst_all.at[pl.ds(nch, 1)])
        plsc.subcore_barrier()

        def outer(o, _):
            for b in range(NBUF):
                i = o * NBUF + b

                @pl.when(o > 0)
                def _wait_prev_scatter():
                    pltpu.make_async_copy(
                        rows[b], acc.at[dst_all.at[i - NBUF]], ssem[b]
                    ).wait()

                pltpu.async_copy(g_hbm.at[src_all.at[i]], rows[b], gsem[b])
            for b in range(NBUF):
                i = o * NBUF + b
                pltpu.make_async_copy(
                    g_hbm.at[src_all.at[i]], rows[b], gsem[b]
                ).wait()
                pltpu.async_copy(rows[b], acc.at[dst_all.at[i]], ssem[b], add=True)
            return 0

        lax.fori_loop(0, nch // NBUF, outer, 0)
        for b in range(NBUF):
            i = nch - NBUF + b
            pltpu.make_async_copy(rows[b], acc.at[dst_all.at[i]], ssem[b]).wait()

        @pl.when(wid < n_extra)
        def _extra():
            pltpu.sync_copy(g_hbm.at[src_all.at[nch]], rows[0])
            pltpu.sync_copy(rows[0], acc.at[dst_all.at[nch]], add=True)

        plsc.subcore_barrier()
        tpq = NS // (128 // d)
        q = sid // tpq
        r0 = (sid % tpq) * rows_per_tile
        pltpu.sync_copy(
            acc.at[pl.ds(sid * rows_per_tile, rows_per_tile)],
            out_hbm.at[cid, pl.ds(r0, rows_per_tile), pl.ds(q * d, d)],
        )

    return scatter_kernel


BLK = 2000


def _mm1_body(x_ref, w_ref, h1_ref):
    h1_ref[...] = jnp.dot(x_ref[...], w_ref[...], preferred_element_type=jnp.float32)


def _unpack(p, d, nrows):
    parts = [p[:, :, q * d:(q + 1) * d] for q in range(128 // d)]
    return jnp.concatenate(parts, axis=1)[:, :nrows]


def _scale1_body(h1_ref, degp_ref, g1_ref, dinv_ref):
    dp = _unpack(degp_ref[...], 16, h1_ref.shape[0])
    deg = 1.0 + dp[0, :, 0:1] + dp[1, :, 0:1]
    dinv = lax.rsqrt(deg)
    dinv_ref[...] = dinv
    g1_ref[...] = h1_ref[...] * dinv


def _mid_body(sp_ref, g_ref, dinv_ref, b_ref, w_ref, o_ref):
    nrows, d = g_ref.shape
    p = _unpack(sp_ref[...], d, nrows)
    s = g_ref[...] + p[0] + p[1]
    dinv = dinv_ref[...]
    a = jnp.maximum(s * dinv + b_ref[...], 0.0)
    o_ref[...] = jnp.dot(a, w_ref[...], preferred_element_type=jnp.float32) * dinv


def _out_body(sp_ref, g_ref, dinv_ref, b_ref, wfc_ref, bfc_ref, o_ref):
    nrows, d = g_ref.shape
    p = _unpack(sp_ref[...], d, nrows)
    s = g_ref[...] + p[0] + p[1]
    a = jnp.maximum(s * dinv_ref[...] + b_ref[...], 0.0)
    h = jnp.dot(a, wfc_ref[...], preferred_element_type=jnp.float32) + bfc_ref[...]
    m = jnp.max(h, axis=1, keepdims=True)
    lse = m + jnp.log(jnp.sum(jnp.exp(h - m), axis=1, keepdims=True))
    o_ref[...] = h - lse


def _rows(shape):
    if len(shape) == 2:
        return pl.BlockSpec((BLK, shape[1]), lambda i: (i, 0))
    return pl.BlockSpec((shape[0], BLK, shape[2]), lambda i: (0, i, 0))


def _full(shape):
    return pl.BlockSpec(shape, lambda i: tuple(0 for _ in shape))


def kernel(x, edge_index, W1, b1, W2, b2, Wfc, bfc):
    n, d_in = x.shape
    e = edge_index.shape[1]
    d1 = W1.shape[1]
    d2 = W2.shape[1]
    assert n % BLK == 0
    grid = (n // BLK,)

    f32 = jnp.float32
    n_pad = ((n + 127) // 128) * 128
    ei3 = edge_index.astype(jnp.int32).reshape(2, e // CHUNK, CHUNK)
    degp = _make_deg_kernel(n_pad, e)(ei3)
    h1 = pl.pallas_call(
        _mm1_body,
        grid=grid,
        in_specs=[_rows((n, d_in)), _full((d_in, d1))],
        out_specs=_rows((n, d1)),
        out_shape=jax.ShapeDtypeStruct((n, d1), f32),
    )(x, W1)
    g1, dinv = pl.pallas_call(
        _scale1_body,
        out_shape=(
            jax.ShapeDtypeStruct((n, d1), f32),
            jax.ShapeDtypeStruct((n, 1), f32),
        ),
    )(h1, degp)
    s1p = _make_scatter_kernel(n, n_pad, e, d1)(g1, ei3)
    g2 = pl.pallas_call(
        _mid_body,
        out_shape=jax.ShapeDtypeStruct((n, d2), f32),
    )(s1p, g1, dinv, b1.reshape(1, d1), W2)
    s2p = _make_scatter_kernel(n, n_pad, e, d2)(g2, ei3)
    out = pl.pallas_call(
        _out_body,
        out_shape=jax.ShapeDtypeStruct((n, 2), f32),
    )(s2p, g2, dinv, b2.reshape(1, d2), Wfc, bfc.reshape(1, 2))
    return out
